# Initial kernel scaffold; baseline (speedup 1.0000x reference)
import jax
import jax.numpy as jnp
from jax.experimental import pallas as pl


def kernel(x, t, pad, tables, temporal_table, norm_g, norm_b, sum_g, sum_b):
    raise NotImplementedError("write your pallas kernel here")


# speedup vs baseline: 1.0862x; 1.0862x over previous
import functools

import jax
import jax.numpy as jnp
from jax import lax
from jax.experimental import pallas as pl
from jax.experimental.pallas import tpu as pltpu
from jax.experimental.pallas import tpu_sc as plsc

B = 4096
F = 26
V = 100000
C = 64
EPS = 1e-5

_info = plsc.get_sparse_core_info()
NC = _info.num_cores
NS = _info.num_subcores
L = _info.num_lanes
NW = NC * NS
BW = B // NW
CL = C // L


def _rsqrt(x):
    i = lax.bitcast_convert_type(x, jnp.int32)
    i = jnp.int32(0x5F3759DF) - lax.shift_right_logical(i, 1)
    y = lax.bitcast_convert_type(i, jnp.float32)
    for _ in range(3):
        y = y * (1.5 - 0.5 * x * y * y)
    return y


@functools.partial(
    pl.kernel,
    out_type=jax.ShapeDtypeStruct((B, C), jnp.float32),
    mesh=plsc.VectorSubcoreMesh(core_axis_name="c", subcore_axis_name="s"),
    compiler_params=pltpu.CompilerParams(needs_layout_passes=False,
                                         use_tc_tiling_on_sc=False),
    scratch_types=[
        pltpu.VMEM((F * BW,), jnp.int32),
        pltpu.VMEM((BW,), jnp.int32),
        pltpu.VMEM((BW, C), jnp.float32),
        pltpu.VMEM((BW, C), jnp.float32),
        pltpu.VMEM((4 * C,), jnp.float32),
        pltpu.SemaphoreType.DMA,
    ],
)
def _sc_embed(tables_hbm, temporal_hbm, xt_hbm, t_hbm, ng_hbm, nb_hbm,
              sg_hbm, sb_hbm, out_hbm,
              idx_v, tw_v, gbuf_v, acc_v, prm_v, sem0):
    cid = lax.axis_index("c")
    sid = lax.axis_index("s")
    wid = sid * NC + cid
    base = wid * BW

    pltpu.sync_copy(t_hbm.at[pl.ds(base, BW)], tw_v)
    tcopy = pltpu.async_copy(temporal_hbm.at[tw_v], acc_v, sem0)

    pltpu.sync_copy(ng_hbm, prm_v.at[pl.ds(0, C)])
    pltpu.sync_copy(nb_hbm, prm_v.at[pl.ds(C, C)])
    pltpu.sync_copy(sg_hbm, prm_v.at[pl.ds(2 * C, C)])
    pltpu.sync_copy(sb_hbm, prm_v.at[pl.ds(3 * C, C)])

    def build_idx(f, _):
        pltpu.sync_copy(xt_hbm.at[f, pl.ds(base, BW)],
                        idx_v.at[pl.ds(f * BW, BW)])
        off = f * V
        for j in range(BW // L):
            s = f * BW + j * L
            idx_v[pl.ds(s, L)] = idx_v[pl.ds(s, L)] + off
        return 0
    lax.fori_loop(0, F, build_idx, 0)

    gvec = [prm_v[pl.ds(u * L, L)] for u in range(CL)]
    bvec = [prm_v[pl.ds(C + u * L, L)] for u in range(CL)]
    sgvec = [prm_v[pl.ds(2 * C + u * L, L)] for u in range(CL)]
    sbvec = [prm_v[pl.ds(3 * C + u * L, L)] for u in range(CL)]

    tcopy.wait()

    def row_ln_update(b, src_ref, gain):
        v = [src_ref[b, pl.ds(u * L, L)] for u in range(CL)]
        sv = (v[0] + v[1]) + (v[2] + v[3])
        qv = (v[0] * v[0] + v[1] * v[1]) + (v[2] * v[2] + v[3] * v[3])
        tot = jnp.sum(sv)
        totq = jnp.sum(qv)
        mu = tot * (1.0 / C)
        var = totq * (1.0 / C) - mu * mu
        muv = jnp.full((L,), mu, jnp.float32)
        rv = _rsqrt(jnp.full((L,), var + EPS, jnp.float32))
        return [gain[u] * ((v[u] - muv) * rv) for u in range(CL)]

    def field_body(f, _):
        pltpu.async_copy(tables_hbm.at[idx_v.at[pl.ds(f * BW, BW)]],
                         gbuf_v, sem0).wait()

        def row_body(b, _):
            upd = row_ln_update(b, gbuf_v, gvec)
            for u in range(CL):
                p = pl.ds(u * L, L)
                acc_v[b, p] = acc_v[b, p] + upd[u]
            return 0
        lax.fori_loop(0, BW, row_body, 0)
        return 0
    lax.fori_loop(0, F, field_body, 0)

    def tail_body(b, _):
        for u in range(CL):
            p = pl.ds(u * L, L)
            acc_v[b, p] = acc_v[b, p] + float(F) * bvec[u]
        upd = row_ln_update(b, acc_v, sgvec)
        for u in range(CL):
            acc_v[b, pl.ds(u * L, L)] = upd[u] + sbvec[u]
        return 0
    lax.fori_loop(0, BW, tail_body, 0)

    pltpu.sync_copy(acc_v, out_hbm.at[pl.ds(base, BW)])


def kernel(x, t, pad, tables, temporal_table, norm_g, norm_b, sum_g, sum_b):
    tables_flat = tables.reshape(F * V, C)
    xt = x.T
    out = _sc_embed(tables_flat, temporal_table, xt, t,
                    norm_g, norm_b, sum_g, sum_b)
    return (out, t, pad)
